# Initial kernel scaffold; baseline (speedup 1.0000x reference)
import jax
import jax.numpy as jnp
from jax.experimental import pallas as pl


def kernel(input_ids, token_type_ids, tok_emb, pos_emb, seg_emb, gamma, beta):
    raise NotImplementedError("write your pallas kernel here")


# speedup vs baseline: 1.9881x; 1.9881x over previous
import functools

import numpy as np

import jax
import jax.numpy as jnp
from jax import lax
from jax.experimental import pallas as pl
from jax.experimental.pallas import tpu as pltpu
from jax.experimental.pallas import tpu_sc as plsc

_B, _L, _V, _H, _MAXLEN = 4096, 200, 1000000, 64, 512
_NW = 32
_ROWS = _B // _NW
_GHALF = _L // 2


def _lane_perm(v, idx):
    return lax.gather(
        v, idx.reshape(16, 1),
        dimension_numbers=lax.GatherDimensionNumbers(
            offset_dims=(), collapsed_slice_dims=(0,), start_index_map=(0,)),
        slice_sizes=(1,),
        mode=lax.GatherScatterMode.PROMISE_IN_BOUNDS)


def _iota16():
    return lax.iota(jnp.int32, 16)


def _lane_splat(v, j):
    return _lane_perm(v, lax.full((16,), j, jnp.int32))


def _newton_rsqrt(a):
    bits = lax.bitcast_convert_type(a, jnp.int32)
    seed = jnp.full_like(bits, 0x5F3759DF) - lax.shift_right_arithmetic(
        bits, jnp.ones_like(bits))
    y = lax.bitcast_convert_type(seed, jnp.float32)
    ah = a * 0.5
    for _ in range(3):
        y = y * (1.5 - ah * y * y)
    return y


def _vsum_splat(v):
    for s in (1, 2, 4, 8):
        v = v + _lane_perm(v, _iota16() ^ s)
    return v


def _sc_kernel(ids_hbm, tt_hbm, tok_hbm, pos_hbm, seg_hbm, gam_hbm, bet_hbm,
               out_hbm, idsv, ttv, tokv, outv, posv, segv, gbv, sem):
    wid = lax.axis_index("s") * 2 + lax.axis_index("c")
    base = wid * _ROWS

    pltpu.sync_copy(pos_hbm.at[pl.ds(0, _L)], posv)
    pltpu.sync_copy(seg_hbm, segv)
    pltpu.sync_copy(gam_hbm, gbv.at[pl.ds(0, _H)])
    pltpu.sync_copy(bet_hbm, gbv.at[pl.ds(_H, _H)])

    s0 = [segv[pl.ds(i * 16, 16)] for i in range(4)]
    sd = [segv[pl.ds(_H + i * 16, 16)] - s0[i] for i in range(4)]
    gv = [gbv[pl.ds(i * 16, 16)] for i in range(4)]
    bv = [gbv[pl.ds(_H + i * 16, 16)] for i in range(4)]

    def fold_body(t, carry):
        for i in range(4):
            sl = pl.ds(i * 16, 16)
            posv[t, sl] = posv[t, sl] + s0[i]
        return carry
    lax.fori_loop(0, _L, fold_body, 0)

    def row_body(c, carry):
        b = base + c
        pltpu.sync_copy(ids_hbm.at[b], idsv)
        pltpu.sync_copy(tt_hbm.at[b], ttv.at[pl.ds(0, _L)])
        cp0 = pltpu.async_copy(tok_hbm.at[idsv.at[0]],
                               tokv.at[pl.ds(0, _GHALF)], sem)
        cp1 = pltpu.async_copy(tok_hbm.at[idsv.at[1]],
                               tokv.at[pl.ds(_GHALF, _GHALF)], sem)
        cp0.wait()
        cp1.wait()

        def group_body(g, carry2):
            ttf = ttv[pl.ds(g * 8, 16)].astype(jnp.float32)
            for j in range(8):
                t = g * 8 + j
                tts = _lane_splat(ttf, j)
                xs = []
                for i in range(4):
                    sl = pl.ds(i * 16, 16)
                    xs.append(tokv[t, sl] + posv[t, sl] + tts * sd[i])
                ssum = _vsum_splat(xs[0] + xs[1] + xs[2] + xs[3])
                qsum = _vsum_splat(xs[0] * xs[0] + xs[1] * xs[1]
                                   + xs[2] * xs[2] + xs[3] * xs[3])
                mu = ssum * (1.0 / _H)
                var = qsum * (1.0 / _H) - mu * mu
                rstd = _newton_rsqrt(var + 1e-5)
                for i in range(4):
                    sl = pl.ds(i * 16, 16)
                    outv[t, sl] = (xs[i] - mu) * rstd * gv[i] + bv[i]
            return carry2
        lax.fori_loop(0, _L // 8, group_body, 0)

        pltpu.sync_copy(outv, out_hbm.at[b])
        return carry
    lax.fori_loop(0, _ROWS, row_body, 0)


def kernel(input_ids, token_type_ids, tok_emb, pos_emb, seg_emb, gamma, beta):
    ids32 = input_ids.astype(jnp.int32).reshape(_B, 2, _GHALF)
    tt32 = token_type_ids.astype(jnp.int32)
    segf = seg_emb.reshape(2 * _H)

    mesh = plsc.VectorSubcoreMesh(core_axis_name="c", subcore_axis_name="s")
    run = pl.kernel(
        _sc_kernel,
        mesh=mesh,
        compiler_params=pltpu.CompilerParams(use_tc_tiling_on_sc=False),
        out_type=jax.ShapeDtypeStruct((_B, _L, _H), jnp.float32),
        scratch_types=[
            pltpu.VMEM((2, _GHALF), jnp.int32),
            pltpu.VMEM((_L + 8,), jnp.int32),
            pltpu.VMEM((_L, _H), jnp.float32),
            pltpu.VMEM((_L, _H), jnp.float32),
            pltpu.VMEM((_L, _H), jnp.float32),
            pltpu.VMEM((2 * _H,), jnp.float32),
            pltpu.VMEM((2 * _H,), jnp.float32),
            pltpu.SemaphoreType.DMA,
        ],
    )
    return run(ids32, tt32, tok_emb, pos_emb, segf, gamma, beta)
